# baseline (speedup 1.0000x reference)
import functools

import jax
import jax.numpy as jnp
from jax import lax
from jax.experimental import pallas as pl
from jax.experimental.pallas import tpu as pltpu
from jax.experimental.pallas import tpu_sc as plsc

DIM = 768
HID = 256
NE = 64
SHID = 512
T = 2048
P = T * 2
BLK = 64
NT = 128
XS_ROWS = NT * BLK

def _gate_body(x_ref, gw_ref, w_ref, idx_ref):
    xb = x_ref[...]
    gw = gw_ref[...]
    logits = lax.dot_general(
        xb, gw, (((1,), (1,)), ((), ())), preferred_element_type=jnp.float32)
    m = jnp.max(logits, axis=1, keepdims=True)
    ex = jnp.exp(logits - m)
    scores = ex / jnp.sum(ex, axis=1, keepdims=True)
    lanes = lax.broadcasted_iota(jnp.int32, scores.shape, 1)
    m1 = jnp.max(scores, axis=1, keepdims=True)
    i1 = jnp.min(jnp.where(scores == m1, lanes, NE), axis=1, keepdims=True)
    s2 = jnp.where(lanes == i1, -jnp.inf, scores)
    m2 = jnp.max(s2, axis=1, keepdims=True)
    i2 = jnp.min(jnp.where(s2 == m2, lanes, NE), axis=1, keepdims=True)
    w_ref[...] = jnp.concatenate([m1, m2], axis=1)
    idx_ref[...] = jnp.concatenate([i1, i2], axis=1).astype(jnp.int32)


def _gate(x2d, gate_w):
    return pl.pallas_call(
        _gate_body,
        out_shape=(
            jax.ShapeDtypeStruct((T, 2), jnp.float32),
            jax.ShapeDtypeStruct((T, 2), jnp.int32),
        ),
    )(x2d, gate_w)


_R_NW = 16
_R_CHUNK = P // _R_NW

def _bcast_elem(ref, j):
    return plsc.load_gather(ref, [jnp.full((16,), j, dtype=jnp.int32)])


def _route_body(idx_hbm, x_hbm, xs_hbm, dst_hbm, eof_hbm, xt_hbm, yt_hbm,
                keys_v, rank_v, cnt_v, pb_v, hist_v, tok_v, dst_v, rows_v,
                eof_v, xt_v, yt_v, hist_sh, sem):
    wid = lax.axis_index("s")
    base_pair = wid * _R_CHUNK
    lanes = lax.iota(jnp.int32, 16)
    zeros16 = jnp.zeros((16,), jnp.int32)

    pltpu.sync_copy(idx_hbm.at[pl.ds(base_pair, _R_CHUNK)], keys_v)
    for b in range(4):
        cnt_v[pl.ds(16 * b, 16)] = zeros16

    def rank_step(i, carry):
        kv = keys_v[pl.ds(i * 16, 16)]
        prior = zeros16
        total = zeros16
        for j in range(16):
            bj = plsc.load_gather(keys_v, [zeros16 + (i * 16 + j)])
            mi = (kv == bj).astype(jnp.int32)
            total = total + mi
            prior = prior + jnp.where(lanes > j, mi, 0)
        cur = plsc.load_gather(cnt_v, [kv])
        rank_v[pl.ds(i * 16, 16)] = cur + prior
        plsc.store_scatter(cnt_v, [kv], cur + total, mask=prior == total - 1)
        return carry

    lax.fori_loop(0, _R_CHUNK // 16, rank_step, 0)

    pltpu.sync_copy(cnt_v, hist_sh.at[pl.ds(wid * NE, NE)])
    plsc.subcore_barrier()
    pltpu.sync_copy(hist_sh, hist_v)

    tot = []
    mybase = []
    for b in range(4):
        t_b = zeros16
        m_b = zeros16
        for w in range(_R_NW):
            h = hist_v[pl.ds(w * NE + 16 * b, 16)]
            t_b = t_b + h
            m_b = m_b + h * (w < wid).astype(jnp.int32)
        tot.append(t_b)
        mybase.append(m_b)

    carry = jnp.int32(0)
    pstart_tiles = []
    for b in range(4):
        nt_b = (tot[b] + (BLK - 1)) // BLK
        inc = plsc.cumsum(nt_b)
        pstart_tiles.append(inc - nt_b + carry)
        carry = carry + jnp.sum(nt_b)

    for b in range(4):
        pb_v[pl.ds(16 * b, 16)] = pstart_tiles[b] * BLK + mybase[b]

    @pl.when(wid == 0)
    def _():
        for t in range(NT // 16):
            eof_v[pl.ds(16 * t, 16)] = zeros16
        for b in range(4):
            nt_b = (tot[b] + (BLK - 1)) // BLK
            idxs = jnp.minimum(pstart_tiles[b], NT - 1)
            plsc.store_scatter(eof_v, [idxs], lanes + 16 * b, mask=nt_b > 0)
        c = jnp.int32(0)
        for t in range(NT // 16):
            v = jnp.maximum(plsc.cummax(eof_v[pl.ds(16 * t, 16)]), c)
            eof_v[pl.ds(16 * t, 16)] = v
            c = jnp.max(v)
        pltpu.sync_copy(eof_v, eof_hbm)
        for t in range(NT // 16):
            tv = lanes + 16 * t
            used = (tv < carry).astype(jnp.int32)
            xt_v[pl.ds(16 * t, 16)] = tv * used
            yt_v[pl.ds(16 * t, 16)] = tv * used + NT * (1 - used)
        pltpu.sync_copy(xt_v, xt_hbm)
        pltpu.sync_copy(yt_v, yt_hbm)

    def dst_step(i, carry):
        kv = keys_v[pl.ds(i * 16, 16)]
        seg = plsc.load_gather(pb_v, [kv])
        d = seg + rank_v[pl.ds(i * 16, 16)]
        h = i // 8
        l = (i % 8) * 16
        dst_v[h, pl.ds(l, 16)] = d
        tok_v[h, pl.ds(l, 16)] = (base_pair + i * 16 + lanes) >> 1
        return carry

    for i in range(_R_CHUNK // 16):
        dst_step(i, 0)

    for h in range(2):
        pltpu.async_copy(x_hbm.at[tok_v.at[h]], rows_v, sem).wait()
        pltpu.async_copy(rows_v, xs_hbm.at[dst_v.at[h]], sem).wait()
        pltpu.sync_copy(dst_v.at[h],
                        dst_hbm.at[pl.ds(base_pair + h * 128, 128)])


def _route(flat_idx, x2d):
    mesh = plsc.VectorSubcoreMesh(
        core_axis_name="c", subcore_axis_name="s", num_cores=1)
    f = functools.partial(
        pl.kernel,
        compiler_params=pltpu.CompilerParams(needs_layout_passes=False),
        out_type=(
            jax.ShapeDtypeStruct((XS_ROWS, DIM), jnp.float32),
            jax.ShapeDtypeStruct((P,), jnp.int32),
            jax.ShapeDtypeStruct((NT,), jnp.int32),
            jax.ShapeDtypeStruct((NT,), jnp.int32),
            jax.ShapeDtypeStruct((NT,), jnp.int32),
        ),
        mesh=mesh,
        scratch_types=[
            pltpu.VMEM((_R_CHUNK,), jnp.int32),
            pltpu.VMEM((_R_CHUNK,), jnp.int32),
            pltpu.VMEM((NE,), jnp.int32),
            pltpu.VMEM((NE,), jnp.int32),
            pltpu.VMEM((_R_NW * NE,), jnp.int32),
            pltpu.VMEM((2, 128), jnp.int32),
            pltpu.VMEM((2, 128), jnp.int32),
            pltpu.VMEM((128, DIM), jnp.float32),
            pltpu.VMEM((NT,), jnp.int32),
            pltpu.VMEM((NT,), jnp.int32),
            pltpu.VMEM((NT,), jnp.int32),
            pltpu.VMEM_SHARED((_R_NW * NE,), jnp.int32),
            pltpu.SemaphoreType.DMA,
        ],
    )(_route_body)
    return f(flat_idx, x2d)


def _silu(a):
    return a / (1.0 + jnp.exp(-a))


def _swiglu_tile(xb, w1, w3, w2):
    xb = xb.astype(jnp.bfloat16)
    a = lax.dot_general(
        xb, w1.astype(jnp.bfloat16), (((1,), (1,)), ((), ())),
        preferred_element_type=jnp.float32)
    b = lax.dot_general(
        xb, w3.astype(jnp.bfloat16), (((1,), (1,)), ((), ())),
        preferred_element_type=jnp.float32)
    return lax.dot_general(
        (_silu(a) * b).astype(jnp.bfloat16), w2.astype(jnp.bfloat16),
        (((1,), (1,)), ((), ())), preferred_element_type=jnp.float32)


def _gemm_body(eof_ref, xt_ref, yt_ref, xs_ref, w1_ref, w3_ref, w2_ref,
               ys_ref):
    del eof_ref, xt_ref, yt_ref
    ys_ref[...] = _swiglu_tile(xs_ref[...], w1_ref[0], w3_ref[0], w2_ref[0])


def _grouped_gemm(eof, xt, yt, xs, W1, W3, W2):
    grid_spec = pltpu.PrefetchScalarGridSpec(
        num_scalar_prefetch=3,
        grid=(NT,),
        in_specs=[
            pl.BlockSpec((BLK, DIM), lambda g, eof, xt, yt: (xt[g], 0)),
            pl.BlockSpec((1, HID, DIM), lambda g, eof, xt, yt: (eof[g], 0, 0)),
            pl.BlockSpec((1, HID, DIM), lambda g, eof, xt, yt: (eof[g], 0, 0)),
            pl.BlockSpec((1, DIM, HID), lambda g, eof, xt, yt: (eof[g], 0, 0)),
        ],
        out_specs=pl.BlockSpec((BLK, DIM), lambda g, eof, xt, yt: (yt[g], 0)),
    )
    return pl.pallas_call(
        _gemm_body,
        grid_spec=grid_spec,
        out_shape=jax.ShapeDtypeStruct((XS_ROWS + BLK, DIM), jnp.float32),
    )(eof, xt, yt, xs, W1, W3, W2)


def _shared_body(x_ref, ws1_ref, ws3_ref, ws2_ref, out_ref):
    out_ref[...] = _swiglu_tile(x_ref[...], ws1_ref[...], ws3_ref[...],
                                ws2_ref[...])


def _shared(x2d, Ws1, Ws3, Ws2):
    blk = 256
    return pl.pallas_call(
        _shared_body,
        grid=(T // blk,),
        in_specs=[
            pl.BlockSpec((blk, DIM), lambda i: (i, 0)),
            pl.BlockSpec((SHID, DIM), lambda i: (0, 0)),
            pl.BlockSpec((SHID, DIM), lambda i: (0, 0)),
            pl.BlockSpec((DIM, SHID), lambda i: (0, 0)),
        ],
        out_specs=pl.BlockSpec((blk, DIM), lambda i: (i, 0)),
        out_shape=jax.ShapeDtypeStruct((T, DIM), jnp.float32),
    )(x2d, Ws1, Ws3, Ws2)


_C_NW = 32
_C_TOK = T // _C_NW


def _combine_body(dst_hbm, w_hbm, ys_hbm, sh_hbm, y_hbm,
                  dst_v, w_v, rows_v, acc_v, sem):
    wid = lax.axis_index("s") * 2 + lax.axis_index("c")
    base_pair = wid * 2 * _C_TOK
    base_tok = wid * _C_TOK

    pltpu.sync_copy(w_hbm.at[pl.ds(base_pair, 128)], w_v.at[pl.ds(0, 128)])
    for h in range(2):
        pltpu.sync_copy(dst_hbm.at[pl.ds(base_pair + h * 64, 64)],
                        dst_v.at[h])
        pltpu.async_copy(ys_hbm.at[dst_v.at[h]], rows_v, sem).wait()
        pltpu.sync_copy(sh_hbm.at[pl.ds(base_tok + h * 32, 32)], acc_v)

        for tt in range(32):
            w0 = _bcast_elem(w_v, h * 64 + 2 * tt)
            w1 = _bcast_elem(w_v, h * 64 + 2 * tt + 1)

            def _col_loop(tt, w0, w1):
                @plsc.parallel_loop(0, DIM, 16, unroll=4)
                def col_step(o):
                    r0 = rows_v[2 * tt, pl.ds(o, 16)]
                    r1 = rows_v[2 * tt + 1, pl.ds(o, 16)]
                    acc_v[tt, pl.ds(o, 16)] += w0 * r0 + w1 * r1

            _col_loop(tt, w0, w1)

        pltpu.sync_copy(acc_v, y_hbm.at[pl.ds(base_tok + h * 32, 32)])


def _combine(dst, wflat, ys, shared_y):
    mesh = plsc.VectorSubcoreMesh(core_axis_name="c", subcore_axis_name="s")
    f = functools.partial(
        pl.kernel,
        compiler_params=pltpu.CompilerParams(needs_layout_passes=False),
        out_type=jax.ShapeDtypeStruct((T, DIM), jnp.float32),
        mesh=mesh,
        scratch_types=[
            pltpu.VMEM((2, 64), jnp.int32),
            pltpu.VMEM((144,), jnp.float32),
            pltpu.VMEM((64, DIM), jnp.float32),
            pltpu.VMEM((32, DIM), jnp.float32),
            pltpu.SemaphoreType.DMA,
        ],
    )(_combine_body)
    return f(dst, wflat, ys, shared_y)


def kernel(x, gate_w, W1, W2, W3, Ws1, Ws2, Ws3):
    Bx, Sx, D = x.shape
    x2d = x.reshape(-1, D)
    shared_y = _shared(x2d, Ws1, Ws3, Ws2)
    cw, cidx = _gate(x2d, gate_w)
    xs, dst, eof, xt, yt = _route(cidx.reshape(P), x2d)
    ys = _grouped_gemm(eof, xt, yt, xs, W1, W3, W2)
    y = _combine(dst, cw.reshape(P), ys, shared_y)
    return y.reshape(Bx, Sx, D)


# speedup vs baseline: 1.1194x; 1.1194x over previous
import functools

import jax
import jax.numpy as jnp
from jax import lax
from jax.experimental import pallas as pl
from jax.experimental.pallas import tpu as pltpu
from jax.experimental.pallas import tpu_sc as plsc

DIM = 768
HID = 256
NE = 64
SHID = 512
T = 2048
P = T * 2
BLK = 128
NT = 96
XS_ROWS = NT * BLK

def _gate_body(x_ref, gw_ref, w_ref, idx_ref):
    xb = x_ref[...]
    gw = gw_ref[...]
    logits = lax.dot_general(
        xb, gw, (((1,), (1,)), ((), ())), preferred_element_type=jnp.float32)
    m = jnp.max(logits, axis=1, keepdims=True)
    ex = jnp.exp(logits - m)
    scores = ex / jnp.sum(ex, axis=1, keepdims=True)
    lanes = lax.broadcasted_iota(jnp.int32, scores.shape, 1)
    m1 = jnp.max(scores, axis=1, keepdims=True)
    i1 = jnp.min(jnp.where(scores == m1, lanes, NE), axis=1, keepdims=True)
    s2 = jnp.where(lanes == i1, -jnp.inf, scores)
    m2 = jnp.max(s2, axis=1, keepdims=True)
    i2 = jnp.min(jnp.where(s2 == m2, lanes, NE), axis=1, keepdims=True)
    w_ref[...] = jnp.concatenate([m1, m2], axis=1)
    idx_ref[...] = jnp.concatenate([i1, i2], axis=1).astype(jnp.int32)


def _gate(x2d, gate_w):
    return pl.pallas_call(
        _gate_body,
        out_shape=(
            jax.ShapeDtypeStruct((T, 2), jnp.float32),
            jax.ShapeDtypeStruct((T, 2), jnp.int32),
        ),
    )(x2d, gate_w)


_R_NW = 16
_R_CHUNK = P // _R_NW

def _bcast_elem(ref, j):
    return plsc.load_gather(ref, [jnp.full((16,), j, dtype=jnp.int32)])


def _route_body(idx_hbm, x_hbm, xs_hbm, dst_hbm, eof_hbm, xt_hbm, yt_hbm,
                keys_v, rank_v, cnt_v, pb_v, hist_v, tok_v, dst_v, rows_v,
                eof_v, xt_v, yt_v, hist_sh, sem):
    wid = lax.axis_index("s")
    base_pair = wid * _R_CHUNK
    lanes = lax.iota(jnp.int32, 16)
    zeros16 = jnp.zeros((16,), jnp.int32)

    pltpu.sync_copy(idx_hbm.at[pl.ds(base_pair, _R_CHUNK)], keys_v)
    for b in range(4):
        cnt_v[pl.ds(16 * b, 16)] = zeros16

    def rank_step(i, carry):
        kv = keys_v[pl.ds(i * 16, 16)]
        prior = zeros16
        total = zeros16
        for j in range(16):
            bj = plsc.load_gather(keys_v, [zeros16 + (i * 16 + j)])
            mi = (kv == bj).astype(jnp.int32)
            total = total + mi
            prior = prior + jnp.where(lanes > j, mi, 0)
        cur = plsc.load_gather(cnt_v, [kv])
        rank_v[pl.ds(i * 16, 16)] = cur + prior
        plsc.store_scatter(cnt_v, [kv], cur + total, mask=prior == total - 1)
        return carry

    lax.fori_loop(0, _R_CHUNK // 16, rank_step, 0)

    pltpu.sync_copy(cnt_v, hist_sh.at[pl.ds(wid * NE, NE)])
    plsc.subcore_barrier()
    pltpu.sync_copy(hist_sh, hist_v)

    tot = []
    mybase = []
    for b in range(4):
        t_b = zeros16
        m_b = zeros16
        for w in range(_R_NW):
            h = hist_v[pl.ds(w * NE + 16 * b, 16)]
            t_b = t_b + h
            m_b = m_b + h * (w < wid).astype(jnp.int32)
        tot.append(t_b)
        mybase.append(m_b)

    carry = jnp.int32(0)
    pstart_tiles = []
    for b in range(4):
        nt_b = (tot[b] + (BLK - 1)) // BLK
        inc = plsc.cumsum(nt_b)
        pstart_tiles.append(inc - nt_b + carry)
        carry = carry + jnp.sum(nt_b)

    for b in range(4):
        pb_v[pl.ds(16 * b, 16)] = pstart_tiles[b] * BLK + mybase[b]

    @pl.when(wid == 0)
    def _():
        for t in range(NT // 16):
            eof_v[pl.ds(16 * t, 16)] = zeros16
        for b in range(4):
            nt_b = (tot[b] + (BLK - 1)) // BLK
            idxs = jnp.minimum(pstart_tiles[b], NT - 1)
            plsc.store_scatter(eof_v, [idxs], lanes + 16 * b, mask=nt_b > 0)
        c = jnp.int32(0)
        for t in range(NT // 16):
            v = jnp.maximum(plsc.cummax(eof_v[pl.ds(16 * t, 16)]), c)
            eof_v[pl.ds(16 * t, 16)] = v
            c = jnp.max(v)
        pltpu.sync_copy(eof_v, eof_hbm)
        for t in range(NT // 16):
            tv = lanes + 16 * t
            used = (tv < carry).astype(jnp.int32)
            xt_v[pl.ds(16 * t, 16)] = tv * used
            yt_v[pl.ds(16 * t, 16)] = tv * used + NT * (1 - used)
        pltpu.sync_copy(xt_v, xt_hbm)
        pltpu.sync_copy(yt_v, yt_hbm)

    def dst_step(i, carry):
        kv = keys_v[pl.ds(i * 16, 16)]
        seg = plsc.load_gather(pb_v, [kv])
        d = seg + rank_v[pl.ds(i * 16, 16)]
        h = i // 8
        l = (i % 8) * 16
        dst_v[h, pl.ds(l, 16)] = d
        tok_v[h, pl.ds(l, 16)] = (base_pair + i * 16 + lanes) >> 1
        return carry

    for i in range(_R_CHUNK // 16):
        dst_step(i, 0)

    for h in range(2):
        pltpu.async_copy(x_hbm.at[tok_v.at[h]], rows_v, sem).wait()
        pltpu.async_copy(rows_v, xs_hbm.at[dst_v.at[h]], sem).wait()
        pltpu.sync_copy(dst_v.at[h],
                        dst_hbm.at[pl.ds(base_pair + h * 128, 128)])


def _route(flat_idx, x2d):
    mesh = plsc.VectorSubcoreMesh(
        core_axis_name="c", subcore_axis_name="s", num_cores=1)
    f = functools.partial(
        pl.kernel,
        compiler_params=pltpu.CompilerParams(needs_layout_passes=False),
        out_type=(
            jax.ShapeDtypeStruct((XS_ROWS, DIM), jnp.float32),
            jax.ShapeDtypeStruct((P,), jnp.int32),
            jax.ShapeDtypeStruct((NT,), jnp.int32),
            jax.ShapeDtypeStruct((NT,), jnp.int32),
            jax.ShapeDtypeStruct((NT,), jnp.int32),
        ),
        mesh=mesh,
        scratch_types=[
            pltpu.VMEM((_R_CHUNK,), jnp.int32),
            pltpu.VMEM((_R_CHUNK,), jnp.int32),
            pltpu.VMEM((NE,), jnp.int32),
            pltpu.VMEM((NE,), jnp.int32),
            pltpu.VMEM((_R_NW * NE,), jnp.int32),
            pltpu.VMEM((2, 128), jnp.int32),
            pltpu.VMEM((2, 128), jnp.int32),
            pltpu.VMEM((128, DIM), jnp.float32),
            pltpu.VMEM((NT,), jnp.int32),
            pltpu.VMEM((NT,), jnp.int32),
            pltpu.VMEM((NT,), jnp.int32),
            pltpu.VMEM_SHARED((_R_NW * NE,), jnp.int32),
            pltpu.SemaphoreType.DMA,
        ],
    )(_route_body)
    return f(flat_idx, x2d)


def _silu(a):
    return a / (1.0 + jnp.exp(-a))


def _swiglu_tile(xb, w1, w3, w2):
    xb = xb.astype(jnp.bfloat16)
    a = lax.dot_general(
        xb, w1.astype(jnp.bfloat16), (((1,), (1,)), ((), ())),
        preferred_element_type=jnp.float32)
    b = lax.dot_general(
        xb, w3.astype(jnp.bfloat16), (((1,), (1,)), ((), ())),
        preferred_element_type=jnp.float32)
    return lax.dot_general(
        (_silu(a) * b).astype(jnp.bfloat16), w2.astype(jnp.bfloat16),
        (((1,), (1,)), ((), ())), preferred_element_type=jnp.float32)


def _gemm_body(eof_ref, xt_ref, yt_ref, xs_ref, w1_ref, w3_ref, w2_ref,
               ys_ref):
    del eof_ref, xt_ref, yt_ref
    ys_ref[...] = _swiglu_tile(xs_ref[...], w1_ref[0], w3_ref[0], w2_ref[0])


def _grouped_gemm(eof, xt, yt, xs, W1, W3, W2):
    grid_spec = pltpu.PrefetchScalarGridSpec(
        num_scalar_prefetch=3,
        grid=(NT,),
        in_specs=[
            pl.BlockSpec((BLK, DIM), lambda g, eof, xt, yt: (xt[g], 0)),
            pl.BlockSpec((1, HID, DIM), lambda g, eof, xt, yt: (eof[g], 0, 0)),
            pl.BlockSpec((1, HID, DIM), lambda g, eof, xt, yt: (eof[g], 0, 0)),
            pl.BlockSpec((1, DIM, HID), lambda g, eof, xt, yt: (eof[g], 0, 0)),
        ],
        out_specs=pl.BlockSpec((BLK, DIM), lambda g, eof, xt, yt: (yt[g], 0)),
    )
    return pl.pallas_call(
        _gemm_body,
        grid_spec=grid_spec,
        out_shape=jax.ShapeDtypeStruct((XS_ROWS + BLK, DIM), jnp.float32),
    )(eof, xt, yt, xs, W1, W3, W2)


def _shared_body(x_ref, ws1_ref, ws3_ref, ws2_ref, out_ref):
    out_ref[...] = _swiglu_tile(x_ref[...], ws1_ref[...], ws3_ref[...],
                                ws2_ref[...])


def _shared(x2d, Ws1, Ws3, Ws2):
    blk = 256
    return pl.pallas_call(
        _shared_body,
        grid=(T // blk,),
        in_specs=[
            pl.BlockSpec((blk, DIM), lambda i: (i, 0)),
            pl.BlockSpec((SHID, DIM), lambda i: (0, 0)),
            pl.BlockSpec((SHID, DIM), lambda i: (0, 0)),
            pl.BlockSpec((DIM, SHID), lambda i: (0, 0)),
        ],
        out_specs=pl.BlockSpec((blk, DIM), lambda i: (i, 0)),
        out_shape=jax.ShapeDtypeStruct((T, DIM), jnp.float32),
    )(x2d, Ws1, Ws3, Ws2)


_C_NW = 32
_C_TOK = T // _C_NW


def _combine_body(dst_hbm, w_hbm, ys_hbm, sh_hbm, y_hbm,
                  dst_v, w_v, rows_v, acc_v, sem):
    wid = lax.axis_index("s") * 2 + lax.axis_index("c")
    base_pair = wid * 2 * _C_TOK
    base_tok = wid * _C_TOK

    pltpu.sync_copy(w_hbm.at[pl.ds(base_pair, 128)], w_v.at[pl.ds(0, 128)])
    for h in range(2):
        pltpu.sync_copy(dst_hbm.at[pl.ds(base_pair + h * 64, 64)],
                        dst_v.at[h])
        pltpu.async_copy(ys_hbm.at[dst_v.at[h]], rows_v, sem).wait()
        pltpu.sync_copy(sh_hbm.at[pl.ds(base_tok + h * 32, 32)], acc_v)

        for tt in range(32):
            w0 = _bcast_elem(w_v, h * 64 + 2 * tt)
            w1 = _bcast_elem(w_v, h * 64 + 2 * tt + 1)

            def _col_loop(tt, w0, w1):
                @plsc.parallel_loop(0, DIM, 16, unroll=4)
                def col_step(o):
                    r0 = rows_v[2 * tt, pl.ds(o, 16)]
                    r1 = rows_v[2 * tt + 1, pl.ds(o, 16)]
                    acc_v[tt, pl.ds(o, 16)] += w0 * r0 + w1 * r1

            _col_loop(tt, w0, w1)

        pltpu.sync_copy(acc_v, y_hbm.at[pl.ds(base_tok + h * 32, 32)])


def _combine(dst, wflat, ys, shared_y):
    mesh = plsc.VectorSubcoreMesh(core_axis_name="c", subcore_axis_name="s")
    f = functools.partial(
        pl.kernel,
        compiler_params=pltpu.CompilerParams(needs_layout_passes=False),
        out_type=jax.ShapeDtypeStruct((T, DIM), jnp.float32),
        mesh=mesh,
        scratch_types=[
            pltpu.VMEM((2, 64), jnp.int32),
            pltpu.VMEM((144,), jnp.float32),
            pltpu.VMEM((64, DIM), jnp.float32),
            pltpu.VMEM((32, DIM), jnp.float32),
            pltpu.SemaphoreType.DMA,
        ],
    )(_combine_body)
    return f(dst, wflat, ys, shared_y)


def kernel(x, gate_w, W1, W2, W3, Ws1, Ws2, Ws3):
    Bx, Sx, D = x.shape
    x2d = x.reshape(-1, D)
    shared_y = _shared(x2d, Ws1, Ws3, Ws2)
    cw, cidx = _gate(x2d, gate_w)
    xs, dst, eof, xt, yt = _route(cidx.reshape(P), x2d)
    ys = _grouped_gemm(eof, xt, yt, xs, W1, W3, W2)
    y = _combine(dst, cw.reshape(P), ys, shared_y)
    return y.reshape(Bx, Sx, D)
